# Initial kernel scaffold; baseline (speedup 1.0000x reference)
import jax
import jax.numpy as jnp
from jax.experimental import pallas as pl


def kernel(x, edge_index, W_emb, b_emb, W_l0, b_l0, W_r0, W_l1, b_l1, W_r1, W_l2, b_l2, W_r2, W_l3, b_l3, W_r3, W_l4, b_l4, W_r4):
    raise NotImplementedError("write your pallas kernel here")


# speedup vs baseline: 2.8626x; 2.8626x over previous
import functools

import jax
import jax.numpy as jnp
from jax import lax
from jax.experimental import pallas as pl
from jax.experimental.pallas import tpu as pltpu
from jax.experimental.pallas import tpu_sc as plsc

_N = 50000
_E = 800000
_NS = 16
_CHUNK = 128
_EPAD = 819200
_ROWS = _EPAD // _CHUNK
_RPT = _ROWS // _NS
_BLK = 80
_NBLK = _RPT // _BLK
_NP = 3
_Q = 8336
_ACC_ROWS = 8448
_STRIPE = _ACC_ROWS // _NS
_LAST = _Q - 15 * _STRIPE
_LAST5 = (_N - 5 * _Q) - 15 * _STRIPE


def _make_sc_agg(with_deg):
  mesh = plsc.VectorSubcoreMesh(core_axis_name="c", subcore_axis_name="s")

  if with_deg:
    out_type = [jax.ShapeDtypeStruct((_N, 128), jnp.float32),
                jax.ShapeDtypeStruct((_N,), jnp.float32)]
  else:
    out_type = jax.ShapeDtypeStruct((_N, 128), jnp.float32)

  scratch = [
      pltpu.VMEM((_BLK, _CHUNK), jnp.int32),
      pltpu.VMEM((_BLK, _CHUNK), jnp.int32),
      pltpu.VMEM((_CHUNK, 128), jnp.float32),
      pltpu.VMEM((_CHUNK, 128), jnp.float32),
      pltpu.VMEM_SHARED((_ACC_ROWS, 128), jnp.float32),
      pltpu.SemaphoreType.DMA,
      pltpu.SemaphoreType.DMA,
  ]
  if with_deg:
    scratch += [
        pltpu.VMEM((_CHUNK,), jnp.float32),
        pltpu.VMEM((_CHUNK,), jnp.float32),
        pltpu.VMEM_SHARED((_ACC_ROWS,), jnp.float32),
    ]

  def body(y_hbm, src_hbm, dst_hbm, zacc_hbm, *rest):
    if with_deg:
      (ones_hbm, zdeg_hbm, agg_hbm, deg_hbm,
       srcv, dstv, g0, g1, accs, sem0, sem1,
       onesv, z1buf, degs) = rest
    else:
      agg_hbm, srcv, dstv, g0, g1, accs, sem0, sem1 = rest

    c = lax.axis_index("c")
    s = lax.axis_index("s")

    if with_deg:
      pltpu.sync_copy(ones_hbm, onesv)

    gb = (g0, g1)
    sems = (sem0, sem1)
    r0 = s * _STRIPE

    for p in range(_NP):
      q = c * _NP + p
      lo = q * _Q
      hi = jnp.minimum(lo + _Q, _N)

      pltpu.sync_copy(zacc_hbm, g0)
      for kz in range(_STRIPE // _CHUNK):
        pltpu.sync_copy(g0, accs.at[pl.ds(r0 + kz * _CHUNK, _CHUNK)])
      ztail = _STRIPE % _CHUNK
      if ztail:
        pltpu.sync_copy(g0.at[pl.ds(0, ztail)],
                        accs.at[pl.ds(r0 + (_STRIPE // _CHUNK) * _CHUNK,
                                      ztail)])
      if with_deg:
        pltpu.sync_copy(zdeg_hbm, z1buf)
        for kz in range(_STRIPE // _CHUNK):
          pltpu.sync_copy(z1buf, degs.at[pl.ds(r0 + kz * _CHUNK, _CHUNK)])
        if ztail:
          pltpu.sync_copy(z1buf.at[pl.ds(0, ztail)],
                          degs.at[pl.ds(r0 + (_STRIPE // _CHUNK) * _CHUNK,
                                        ztail)])

      plsc.subcore_barrier()

      for blk in range(_NBLK):
        base = s * _RPT + blk * _BLK
        pltpu.sync_copy(src_hbm.at[pl.ds(base, _BLK)], srcv)
        pltpu.sync_copy(dst_hbm.at[pl.ds(base, _BLK)], dstv)

        def localize(j, carry):
          for kk in range(_CHUNK // 16):
            sl = pl.ds(kk * 16, 16)
            sv = srcv[j, sl]
            dv = dstv[j, sl]
            mine = jnp.logical_and(dv >= lo, dv < hi)
            srcv[j, sl] = jnp.where(mine, sv, -1)
            dstv[j, sl] = jnp.where(mine, dv - lo, -1)
          return carry

        lax.fori_loop(0, _BLK, localize, 0)

        def _gidx(j):
          return plsc.Indices(srcv.at[j], ignored_value=-1)

        for b in range(2):
          pltpu.async_copy(y_hbm.at[_gidx(b)], gb[b], sems[b])

        def chunk_pair(t, carry):
          for b in range(2):
            j = t * 2 + b
            pltpu.make_async_copy(y_hbm.at[_gidx(j)], gb[b], sems[b]).wait()
            didx = plsc.Indices(dstv.at[j], ignored_value=-1)
            pltpu.sync_copy(gb[b], accs.at[didx], add=True)
            if with_deg:
              pltpu.sync_copy(onesv, degs.at[didx], add=True)
            nxt = j + 2

            @pl.when(nxt < _BLK)
            def _():
              pltpu.async_copy(y_hbm.at[_gidx(nxt)], gb[b], sems[b])
          return carry

        lax.fori_loop(0, _BLK // 2, chunk_pair, 0)

      plsc.subcore_barrier()

      @pl.when(s < _NS - 1)
      def _():
        pltpu.sync_copy(accs.at[pl.ds(r0, _STRIPE)],
                        agg_hbm.at[pl.ds(lo + r0, _STRIPE)])

      if p < _NP - 1:
        @pl.when(s == _NS - 1)
        def _():
          pltpu.sync_copy(accs.at[pl.ds(r0, _LAST)],
                          agg_hbm.at[pl.ds(lo + r0, _LAST)])
      else:
        @pl.when(jnp.logical_and(s == _NS - 1, c == 0))
        def _():
          pltpu.sync_copy(accs.at[pl.ds(r0, _LAST)],
                          agg_hbm.at[pl.ds(lo + r0, _LAST)])

        @pl.when(jnp.logical_and(s == _NS - 1, c == 1))
        def _():
          pltpu.sync_copy(accs.at[pl.ds(r0, _LAST5)],
                          agg_hbm.at[pl.ds(lo + r0, _LAST5)])

      if with_deg:
        def _deg_out(nrows):
          for kz in range(nrows // _CHUNK):
            off = r0 + kz * _CHUNK
            pltpu.sync_copy(degs.at[pl.ds(off, _CHUNK)], z1buf)
            pltpu.sync_copy(z1buf, deg_hbm.at[pl.ds(lo + off, _CHUNK)])
          tail = nrows % _CHUNK
          if tail:
            off = r0 + (nrows // _CHUNK) * _CHUNK
            pltpu.sync_copy(degs.at[pl.ds(off, tail)],
                            z1buf.at[pl.ds(0, tail)])
            pltpu.sync_copy(z1buf.at[pl.ds(0, tail)],
                            deg_hbm.at[pl.ds(lo + off, tail)])

        @pl.when(s < _NS - 1)
        def _():
          _deg_out(_STRIPE)

        if p < _NP - 1:
          @pl.when(s == _NS - 1)
          def _():
            _deg_out(_LAST)
        else:
          @pl.when(jnp.logical_and(s == _NS - 1, c == 0))
          def _():
            _deg_out(_LAST)

          @pl.when(jnp.logical_and(s == _NS - 1, c == 1))
          def _():
            _deg_out(_LAST5)

  return pl.kernel(body, out_type=out_type, mesh=mesh, scratch_types=scratch)


_B = 2000


def _emb_body(x_ref, wemb_ref, bemb_ref, wl0_ref, h_ref, y_ref):
  h = jnp.maximum(
      jnp.dot(x_ref[...], wemb_ref[...], preferred_element_type=jnp.float32)
      + bemb_ref[...], 0.0)
  h_ref[...] = h
  y_ref[...] = jnp.dot(h, wl0_ref[...], preferred_element_type=jnp.float32)


def _emb_call(x, wemb, bemb, wl0p):
  return pl.pallas_call(
      _emb_body,
      grid=(_N // _B,),
      in_specs=[
          pl.BlockSpec((_B, 100), lambda i: (i, 0)),
          pl.BlockSpec((100, 128), lambda i: (0, 0)),
          pl.BlockSpec((1, 128), lambda i: (0, 0)),
          pl.BlockSpec((128, 128), lambda i: (0, 0)),
      ],
      out_specs=[
          pl.BlockSpec((_B, 128), lambda i: (i, 0)),
          pl.BlockSpec((_B, 128), lambda i: (i, 0)),
      ],
      out_shape=[
          jax.ShapeDtypeStruct((_N, 128), jnp.float32),
          jax.ShapeDtypeStruct((_N, 128), jnp.float32),
      ],
  )(x, wemb, bemb, wl0p)


def _mid_body(agg_ref, deg_ref, h_ref, wr_ref, bl_ref, wln_ref,
              h_out_ref, y_ref):
  deg = jnp.maximum(deg_ref[...], 1.0)
  mean = agg_ref[...][:, :80] / deg
  t = (mean + bl_ref[...]
       + jnp.dot(h_ref[...], wr_ref[...], preferred_element_type=jnp.float32))
  nrm = jnp.sqrt(jnp.sum(t * t, axis=1, keepdims=True))
  hn = jnp.maximum(t / jnp.maximum(nrm, 1e-12), 0.0)
  h_out_ref[...] = hn
  y_ref[...] = jnp.dot(hn, wln_ref[...], preferred_element_type=jnp.float32)


def _mid_call(agg, deg2d, h, wr, bl, wlnp, din):
  return pl.pallas_call(
      _mid_body,
      grid=(_N // _B,),
      in_specs=[
          pl.BlockSpec((_B, 128), lambda i: (i, 0)),
          pl.BlockSpec((_B, 1), lambda i: (i, 0)),
          pl.BlockSpec((_B, din), lambda i: (i, 0)),
          pl.BlockSpec((din, 80), lambda i: (0, 0)),
          pl.BlockSpec((1, 80), lambda i: (0, 0)),
          pl.BlockSpec((80, 128), lambda i: (0, 0)),
      ],
      out_specs=[
          pl.BlockSpec((_B, 80), lambda i: (i, 0)),
          pl.BlockSpec((_B, 128), lambda i: (i, 0)),
      ],
      out_shape=[
          jax.ShapeDtypeStruct((_N, 80), jnp.float32),
          jax.ShapeDtypeStruct((_N, 128), jnp.float32),
      ],
  )(agg, deg2d, h, wr, bl, wlnp)


def _final_body(agg_ref, deg_ref, h_ref, wr_ref, bl_ref, out_ref):
  deg = jnp.maximum(deg_ref[...], 1.0)
  mean = agg_ref[...][:, :18] / deg
  t = (mean + bl_ref[...]
       + jnp.dot(h_ref[...], wr_ref[...], preferred_element_type=jnp.float32))
  nrm = jnp.sqrt(jnp.sum(t * t, axis=1, keepdims=True))
  out_ref[...] = t / jnp.maximum(nrm, 1e-12)


def _final_call(agg, deg2d, h, wr, bl):
  return pl.pallas_call(
      _final_body,
      grid=(_N // _B,),
      in_specs=[
          pl.BlockSpec((_B, 128), lambda i: (i, 0)),
          pl.BlockSpec((_B, 1), lambda i: (i, 0)),
          pl.BlockSpec((_B, 80), lambda i: (i, 0)),
          pl.BlockSpec((80, 18), lambda i: (0, 0)),
          pl.BlockSpec((1, 18), lambda i: (0, 0)),
      ],
      out_specs=pl.BlockSpec((_B, 18), lambda i: (i, 0)),
      out_shape=jax.ShapeDtypeStruct((_N, 18), jnp.float32),
  )(agg, deg2d, h, wr, bl)


def _pad_cols(w, cols):
  return jnp.pad(w, ((0, 0), (0, cols - w.shape[1])))


def kernel(x, edge_index, W_emb, b_emb,
           W_l0, b_l0, W_r0,
           W_l1, b_l1, W_r1,
           W_l2, b_l2, W_r2,
           W_l3, b_l3, W_r3,
           W_l4, b_l4, W_r4):
  pad = _EPAD - _E
  srcp = jnp.concatenate(
      [edge_index[0], jnp.zeros((pad,), jnp.int32)]).reshape(_ROWS, _CHUNK)
  dstp = jnp.concatenate(
      [edge_index[1], jnp.full((pad,), _N, jnp.int32)]).reshape(_ROWS, _CHUNK)

  zacc = jnp.zeros((_CHUNK, 128), jnp.float32)
  zdeg = jnp.zeros((_CHUNK,), jnp.float32)
  ones = jnp.ones((_CHUNK,), jnp.float32)

  wl0p = _pad_cols(W_l0, 128)
  wl1p = _pad_cols(W_l1, 128)
  wl2p = _pad_cols(W_l2, 128)
  wl3p = _pad_cols(W_l3, 128)
  wl4p = _pad_cols(W_l4, 128)

  sc = _make_sc_agg(True)

  h0, y0 = _emb_call(x, W_emb, b_emb.reshape(1, 128), wl0p)
  agg0, deg = sc(y0, srcp, dstp, zacc, ones, zdeg)
  deg2d = deg.reshape(_N, 1)

  h1, y1 = _mid_call(agg0, deg2d, h0, W_r0, b_l0.reshape(1, 80), wl1p, 128)
  agg1, _ = sc(y1, srcp, dstp, zacc, ones, zdeg)
  h2, y2 = _mid_call(agg1, deg2d, h1, W_r1, b_l1.reshape(1, 80), wl2p, 80)
  agg2, _ = sc(y2, srcp, dstp, zacc, ones, zdeg)
  h3, y3 = _mid_call(agg2, deg2d, h2, W_r2, b_l2.reshape(1, 80), wl3p, 80)
  agg3, _ = sc(y3, srcp, dstp, zacc, ones, zdeg)
  h4, y4 = _mid_call(agg3, deg2d, h3, W_r3, b_l3.reshape(1, 80), wl4p, 80)
  agg4, _ = sc(y4, srcp, dstp, zacc, ones, zdeg)

  return _final_call(agg4, deg2d, h4, W_r4, b_l4.reshape(1, 18))


# speedup vs baseline: 2.9853x; 1.0429x over previous
import functools

import jax
import jax.numpy as jnp
from jax import lax
from jax.experimental import pallas as pl
from jax.experimental.pallas import tpu as pltpu
from jax.experimental.pallas import tpu_sc as plsc

_N = 50000
_E = 800000
_NS = 16
_CHUNK = 128
_EPAD = 819200
_ROWS = _EPAD // _CHUNK
_RPT = _ROWS // _NS
_BLK = 80
_NBLK = _RPT // _BLK
_NP = 3
_Q = 8336
_ACC_ROWS = 8448
_STRIPE = _ACC_ROWS // _NS
_LAST = _Q - 15 * _STRIPE
_LAST5 = (_N - 5 * _Q) - 15 * _STRIPE


def _make_sc_agg(with_deg):
  mesh = plsc.VectorSubcoreMesh(core_axis_name="c", subcore_axis_name="s")

  if with_deg:
    out_type = [jax.ShapeDtypeStruct((_N, 128), jnp.float32),
                jax.ShapeDtypeStruct((_N,), jnp.float32)]
  else:
    out_type = jax.ShapeDtypeStruct((_N, 128), jnp.float32)

  scratch = [
      pltpu.VMEM((_BLK, _CHUNK), jnp.int32),
      pltpu.VMEM((_BLK, _CHUNK), jnp.int32),
      pltpu.VMEM((_CHUNK, 128), jnp.float32),
      pltpu.VMEM((_CHUNK, 128), jnp.float32),
      pltpu.VMEM_SHARED((_ACC_ROWS, 128), jnp.float32),
      pltpu.SemaphoreType.DMA,
      pltpu.SemaphoreType.DMA,
  ]
  if with_deg:
    scratch += [
        pltpu.VMEM((_CHUNK,), jnp.float32),
        pltpu.VMEM((_CHUNK,), jnp.float32),
        pltpu.VMEM_SHARED((_ACC_ROWS,), jnp.float32),
    ]

  def body(y_hbm, src_hbm, dst_hbm, zacc_hbm, *rest):
    if with_deg:
      (ones_hbm, zdeg_hbm, agg_hbm, deg_hbm,
       srcv, dstv, g0, g1, accs, sem0, sem1,
       onesv, z1buf, degs) = rest
    else:
      agg_hbm, srcv, dstv, g0, g1, accs, sem0, sem1 = rest

    c = lax.axis_index("c")
    s = lax.axis_index("s")

    if with_deg:
      pltpu.sync_copy(ones_hbm, onesv)

    gb = (g0, g1)
    sems = (sem0, sem1)
    r0 = s * _STRIPE

    for p in range(_NP):
      q = c * _NP + p
      lo = q * _Q
      hi = jnp.minimum(lo + _Q, _N)

      pltpu.sync_copy(zacc_hbm, g0)
      for kz in range(_STRIPE // _CHUNK):
        pltpu.sync_copy(g0, accs.at[pl.ds(r0 + kz * _CHUNK, _CHUNK)])
      ztail = _STRIPE % _CHUNK
      if ztail:
        pltpu.sync_copy(g0.at[pl.ds(0, ztail)],
                        accs.at[pl.ds(r0 + (_STRIPE // _CHUNK) * _CHUNK,
                                      ztail)])
      if with_deg:
        pltpu.sync_copy(zdeg_hbm, z1buf)
        for kz in range(_STRIPE // _CHUNK):
          pltpu.sync_copy(z1buf, degs.at[pl.ds(r0 + kz * _CHUNK, _CHUNK)])
        if ztail:
          pltpu.sync_copy(z1buf.at[pl.ds(0, ztail)],
                          degs.at[pl.ds(r0 + (_STRIPE // _CHUNK) * _CHUNK,
                                        ztail)])

      plsc.subcore_barrier()

      for blk in range(_NBLK):
        base = s * _RPT + blk * _BLK
        pltpu.sync_copy(src_hbm.at[pl.ds(base, _BLK)], srcv)
        pltpu.sync_copy(dst_hbm.at[pl.ds(base, _BLK)], dstv)

        def localize(j, carry):
          for kk in range(_CHUNK // 16):
            sl = pl.ds(kk * 16, 16)
            sv = srcv[j, sl]
            dv = dstv[j, sl]
            mine = jnp.logical_and(dv >= lo, dv < hi)
            srcv[j, sl] = jnp.where(mine, sv, -1)
            dstv[j, sl] = jnp.where(mine, dv - lo, -1)
          return carry

        lax.fori_loop(0, _BLK, localize, 0)

        def _gidx(j):
          return plsc.Indices(srcv.at[j], ignored_value=-1)

        for b in range(2):
          pltpu.async_copy(y_hbm.at[_gidx(b)], gb[b], sems[b])

        def chunk_pair(t, carry):
          for b in range(2):
            j = t * 2 + b
            pltpu.make_async_copy(y_hbm.at[_gidx(j)], gb[b], sems[b]).wait()
            didx = plsc.Indices(dstv.at[j], ignored_value=-1)
            pltpu.sync_copy(gb[b], accs.at[didx], add=True)
            if with_deg:
              pltpu.sync_copy(onesv, degs.at[didx], add=True)
            nxt = j + 2

            @pl.when(nxt < _BLK)
            def _():
              pltpu.async_copy(y_hbm.at[_gidx(nxt)], gb[b], sems[b])
          return carry

        lax.fori_loop(0, _BLK // 2, chunk_pair, 0)

      plsc.subcore_barrier()

      @pl.when(s < _NS - 1)
      def _():
        pltpu.sync_copy(accs.at[pl.ds(r0, _STRIPE)],
                        agg_hbm.at[pl.ds(lo + r0, _STRIPE)])

      if p < _NP - 1:
        @pl.when(s == _NS - 1)
        def _():
          pltpu.sync_copy(accs.at[pl.ds(r0, _LAST)],
                          agg_hbm.at[pl.ds(lo + r0, _LAST)])
      else:
        @pl.when(jnp.logical_and(s == _NS - 1, c == 0))
        def _():
          pltpu.sync_copy(accs.at[pl.ds(r0, _LAST)],
                          agg_hbm.at[pl.ds(lo + r0, _LAST)])

        @pl.when(jnp.logical_and(s == _NS - 1, c == 1))
        def _():
          pltpu.sync_copy(accs.at[pl.ds(r0, _LAST5)],
                          agg_hbm.at[pl.ds(lo + r0, _LAST5)])

      if with_deg:
        def _deg_out(nrows):
          for kz in range(nrows // _CHUNK):
            off = r0 + kz * _CHUNK
            pltpu.sync_copy(degs.at[pl.ds(off, _CHUNK)], z1buf)
            pltpu.sync_copy(z1buf, deg_hbm.at[pl.ds(lo + off, _CHUNK)])
          tail = nrows % _CHUNK
          if tail:
            off = r0 + (nrows // _CHUNK) * _CHUNK
            pltpu.sync_copy(degs.at[pl.ds(off, tail)],
                            z1buf.at[pl.ds(0, tail)])
            pltpu.sync_copy(z1buf.at[pl.ds(0, tail)],
                            deg_hbm.at[pl.ds(lo + off, tail)])

        @pl.when(s < _NS - 1)
        def _():
          _deg_out(_STRIPE)

        if p < _NP - 1:
          @pl.when(s == _NS - 1)
          def _():
            _deg_out(_LAST)
        else:
          @pl.when(jnp.logical_and(s == _NS - 1, c == 0))
          def _():
            _deg_out(_LAST)

          @pl.when(jnp.logical_and(s == _NS - 1, c == 1))
          def _():
            _deg_out(_LAST5)

  return pl.kernel(body, out_type=out_type, mesh=mesh, scratch_types=scratch)


_B = 2000


def _emb_body(x_ref, wemb_ref, bemb_ref, wl0_ref, h_ref, y_ref):
  h = jnp.maximum(
      jnp.dot(x_ref[...], wemb_ref[...], preferred_element_type=jnp.float32)
      + bemb_ref[...], 0.0)
  h_ref[...] = h
  y_ref[...] = jnp.dot(h, wl0_ref[...], preferred_element_type=jnp.float32)


def _emb_call(x, wemb, bemb, wl0p):
  return pl.pallas_call(
      _emb_body,
      grid=(_N // _B,),
      in_specs=[
          pl.BlockSpec((_B, 100), lambda i: (i, 0)),
          pl.BlockSpec((100, 128), lambda i: (0, 0)),
          pl.BlockSpec((1, 128), lambda i: (0, 0)),
          pl.BlockSpec((128, 128), lambda i: (0, 0)),
      ],
      out_specs=[
          pl.BlockSpec((_B, 128), lambda i: (i, 0)),
          pl.BlockSpec((_B, 128), lambda i: (i, 0)),
      ],
      out_shape=[
          jax.ShapeDtypeStruct((_N, 128), jnp.float32),
          jax.ShapeDtypeStruct((_N, 128), jnp.float32),
      ],
  )(x, wemb, bemb, wl0p)


def _mid_body(agg_ref, deg_ref, h_ref, wr_ref, bl_ref, wln_ref,
              h_out_ref, y_ref):
  deg = jnp.maximum(deg_ref[...], 1.0)
  mean = agg_ref[...][:, :80] / deg
  t = (mean + bl_ref[...]
       + jnp.dot(h_ref[...], wr_ref[...], preferred_element_type=jnp.float32))
  nrm = jnp.sqrt(jnp.sum(t * t, axis=1, keepdims=True))
  hn = jnp.maximum(t / jnp.maximum(nrm, 1e-12), 0.0)
  h_out_ref[...] = hn
  y_ref[...] = jnp.dot(hn, wln_ref[...], preferred_element_type=jnp.float32)


def _mid_call(agg, deg2d, h, wr, bl, wlnp, din):
  return pl.pallas_call(
      _mid_body,
      grid=(_N // _B,),
      in_specs=[
          pl.BlockSpec((_B, 128), lambda i: (i, 0)),
          pl.BlockSpec((_B, 1), lambda i: (i, 0)),
          pl.BlockSpec((_B, din), lambda i: (i, 0)),
          pl.BlockSpec((din, 80), lambda i: (0, 0)),
          pl.BlockSpec((1, 80), lambda i: (0, 0)),
          pl.BlockSpec((80, 128), lambda i: (0, 0)),
      ],
      out_specs=[
          pl.BlockSpec((_B, 80), lambda i: (i, 0)),
          pl.BlockSpec((_B, 128), lambda i: (i, 0)),
      ],
      out_shape=[
          jax.ShapeDtypeStruct((_N, 80), jnp.float32),
          jax.ShapeDtypeStruct((_N, 128), jnp.float32),
      ],
  )(agg, deg2d, h, wr, bl, wlnp)


def _final_body(agg_ref, deg_ref, h_ref, wr_ref, bl_ref, out_ref):
  deg = jnp.maximum(deg_ref[...], 1.0)
  mean = agg_ref[...][:, :18] / deg
  t = (mean + bl_ref[...]
       + jnp.dot(h_ref[...], wr_ref[...], preferred_element_type=jnp.float32))
  nrm = jnp.sqrt(jnp.sum(t * t, axis=1, keepdims=True))
  out_ref[...] = t / jnp.maximum(nrm, 1e-12)


def _final_call(agg, deg2d, h, wr, bl):
  return pl.pallas_call(
      _final_body,
      grid=(_N // _B,),
      in_specs=[
          pl.BlockSpec((_B, 128), lambda i: (i, 0)),
          pl.BlockSpec((_B, 1), lambda i: (i, 0)),
          pl.BlockSpec((_B, 80), lambda i: (i, 0)),
          pl.BlockSpec((80, 18), lambda i: (0, 0)),
          pl.BlockSpec((1, 18), lambda i: (0, 0)),
      ],
      out_specs=pl.BlockSpec((_B, 18), lambda i: (i, 0)),
      out_shape=jax.ShapeDtypeStruct((_N, 18), jnp.float32),
  )(agg, deg2d, h, wr, bl)


def _pad_cols(w, cols):
  return jnp.pad(w, ((0, 0), (0, cols - w.shape[1])))


def kernel(x, edge_index, W_emb, b_emb,
           W_l0, b_l0, W_r0,
           W_l1, b_l1, W_r1,
           W_l2, b_l2, W_r2,
           W_l3, b_l3, W_r3,
           W_l4, b_l4, W_r4):
  pad = _EPAD - _E
  srcp = jnp.concatenate(
      [edge_index[0], jnp.zeros((pad,), jnp.int32)]).reshape(_ROWS, _CHUNK)
  dstp = jnp.concatenate(
      [edge_index[1], jnp.full((pad,), _N, jnp.int32)]).reshape(_ROWS, _CHUNK)

  zacc = jnp.zeros((_CHUNK, 128), jnp.float32)
  zdeg = jnp.zeros((_CHUNK,), jnp.float32)
  ones = jnp.ones((_CHUNK,), jnp.float32)

  wl0p = _pad_cols(W_l0, 128)
  wl1p = _pad_cols(W_l1, 128)
  wl2p = _pad_cols(W_l2, 128)
  wl3p = _pad_cols(W_l3, 128)
  wl4p = _pad_cols(W_l4, 128)

  sc_deg = _make_sc_agg(True)
  sc = _make_sc_agg(False)

  h0, y0 = _emb_call(x, W_emb, b_emb.reshape(1, 128), wl0p)
  agg0, deg = sc_deg(y0, srcp, dstp, zacc, ones, zdeg)
  deg2d = deg.reshape(_N, 1)

  h1, y1 = _mid_call(agg0, deg2d, h0, W_r0, b_l0.reshape(1, 80), wl1p, 128)
  agg1 = sc(y1, srcp, dstp, zacc)
  h2, y2 = _mid_call(agg1, deg2d, h1, W_r1, b_l1.reshape(1, 80), wl2p, 80)
  agg2 = sc(y2, srcp, dstp, zacc)
  h3, y3 = _mid_call(agg2, deg2d, h2, W_r2, b_l2.reshape(1, 80), wl3p, 80)
  agg3 = sc(y3, srcp, dstp, zacc)
  h4, y4 = _mid_call(agg3, deg2d, h3, W_r3, b_l3.reshape(1, 80), wl4p, 80)
  agg4 = sc(y4, srcp, dstp, zacc)

  return _final_call(agg4, deg2d, h4, W_r4, b_l4.reshape(1, 18))


# speedup vs baseline: 5.9724x; 2.0006x over previous
import functools

import jax
import jax.numpy as jnp
from jax import lax
from jax.experimental import pallas as pl
from jax.experimental.pallas import tpu as pltpu
from jax.experimental.pallas import tpu_sc as plsc

_N = 50000
_E = 800000
_NS = 16
_CHUNK = 128
_EPAD = 835584
_ROWS = _EPAD // _CHUNK
_RPT = _ROWS // _NS
_BLK = 24
_NBLK = _RPT // _BLK
_NP = 3
_Q = 8336
_ACC_ROWS = 8448
_STRIPE = _ACC_ROWS // _NS
_LAST = _Q - 15 * _STRIPE
_LAST5 = (_N - 5 * _Q) - 15 * _STRIPE
_CAP = 3200


def _make_sc_agg(with_deg):
  mesh = plsc.VectorSubcoreMesh(core_axis_name="c", subcore_axis_name="s")

  if with_deg:
    out_type = [jax.ShapeDtypeStruct((_N, 128), jnp.float32),
                jax.ShapeDtypeStruct((_N,), jnp.float32)]
  else:
    out_type = jax.ShapeDtypeStruct((_N, 128), jnp.float32)

  scratch = [
      pltpu.VMEM((_BLK, _CHUNK), jnp.int32),
      pltpu.VMEM((_BLK, _CHUNK), jnp.int32),
      pltpu.VMEM((_CAP,), jnp.int32),
      pltpu.VMEM((_CAP,), jnp.int32),
      pltpu.VMEM((16,), jnp.int32),
      pltpu.VMEM((_CHUNK, 128), jnp.float32),
      pltpu.VMEM((_CHUNK, 128), jnp.float32),
      pltpu.VMEM_SHARED((_ACC_ROWS, 128), jnp.float32),
      pltpu.SemaphoreType.DMA,
      pltpu.SemaphoreType.DMA,
  ]
  if with_deg:
    scratch += [
        pltpu.VMEM((_CHUNK,), jnp.float32),
        pltpu.VMEM((_CHUNK,), jnp.float32),
        pltpu.VMEM_SHARED((_ACC_ROWS,), jnp.float32),
    ]

  def body(y_hbm, src_hbm, dst_hbm, zacc_hbm, *rest):
    if with_deg:
      (ones_hbm, zdeg_hbm, agg_hbm, deg_hbm,
       srcv, dstv, csrc, cdst, pcbuf, g0, g1, accs, sem0, sem1,
       onesv, z1buf, degs) = rest
    else:
      agg_hbm, srcv, dstv, csrc, cdst, pcbuf, g0, g1, accs, sem0, sem1 = rest

    c = lax.axis_index("c")
    s = lax.axis_index("s")

    if with_deg:
      pltpu.sync_copy(ones_hbm, onesv)

    gb = (g0, g1)
    sems = (sem0, sem1)
    r0 = s * _STRIPE

    for p in range(_NP):
      q = c * _NP + p
      lo = q * _Q
      hi = jnp.minimum(lo + _Q, _N)

      pltpu.sync_copy(zacc_hbm, g0)
      for kz in range(_STRIPE // _CHUNK):
        pltpu.sync_copy(g0, accs.at[pl.ds(r0 + kz * _CHUNK, _CHUNK)])
      ztail = _STRIPE % _CHUNK
      if ztail:
        pltpu.sync_copy(g0.at[pl.ds(0, ztail)],
                        accs.at[pl.ds(r0 + (_STRIPE // _CHUNK) * _CHUNK,
                                      ztail)])
      if with_deg:
        pltpu.sync_copy(zdeg_hbm, z1buf)
        for kz in range(_STRIPE // _CHUNK):
          pltpu.sync_copy(z1buf, degs.at[pl.ds(r0 + kz * _CHUNK, _CHUNK)])
        if ztail:
          pltpu.sync_copy(z1buf.at[pl.ds(0, ztail)],
                          degs.at[pl.ds(r0 + (_STRIPE // _CHUNK) * _CHUNK,
                                        ztail)])

      plsc.subcore_barrier()

      def _cidx(ref, k):
        return plsc.Indices(ref.at[pl.ds(k * _CHUNK, _CHUNK)],
                            ignored_value=-1)

      def drain(n):
        @pl.when(n > 0)
        def _():
          pltpu.async_copy(y_hbm.at[_cidx(csrc, 0)], g0, sem0)

        @pl.when(n > 1)
        def _():
          pltpu.async_copy(y_hbm.at[_cidx(csrc, 1)], g1, sem1)

        def pair(t, carry):
          j0 = t * 2
          pltpu.make_async_copy(y_hbm.at[_cidx(csrc, j0)], g0, sem0).wait()
          pltpu.sync_copy(g0, accs.at[_cidx(cdst, j0)], add=True)
          if with_deg:
            pltpu.sync_copy(onesv, degs.at[_cidx(cdst, j0)], add=True)

          @pl.when(j0 + 2 < n)
          def _():
            pltpu.async_copy(y_hbm.at[_cidx(csrc, j0 + 2)], g0, sem0)

          j1 = j0 + 1

          @pl.when(j1 < n)
          def _():
            pltpu.make_async_copy(y_hbm.at[_cidx(csrc, j1)], g1, sem1).wait()
            pltpu.sync_copy(g1, accs.at[_cidx(cdst, j1)], add=True)
            if with_deg:
              pltpu.sync_copy(onesv, degs.at[_cidx(cdst, j1)], add=True)

            @pl.when(j1 + 2 < n)
            def _():
              pltpu.async_copy(y_hbm.at[_cidx(csrc, j1 + 2)], g1, sem1)

          return carry

        lax.fori_loop(0, (n + 1) // 2, pair, 0)

      def block_body(blk, cptr):
        base = s * _RPT + blk * _BLK
        pltpu.sync_copy(src_hbm.at[pl.ds(base, _BLK)], srcv)
        pltpu.sync_copy(dst_hbm.at[pl.ds(base, _BLK)], dstv)

        def row(j, cp):
          for kk in range(_CHUNK // 16):
            sl = pl.ds(kk * 16, 16)
            sv = srcv[j, sl]
            dv = dstv[j, sl]
            mine = jnp.logical_and(dv >= lo, dv < hi)
            mi = jnp.where(mine, 1, 0)
            _, ssrc = plsc.sort_key_val(mi, jnp.where(mine, sv, -1),
                                        descending=True)
            _, sdst = plsc.sort_key_val(mi, jnp.where(mine, dv - lo, -1),
                                        descending=True)
            csrc[pl.ds(cp, 16)] = ssrc
            cdst[pl.ds(cp, 16)] = sdst
            cp = cp + plsc.all_reduce_population_count(mine)[0]
          return cp

        cptr = lax.fori_loop(0, _BLK, row, cptr)
        nfull = cptr // _CHUNK
        drain(nfull)
        rem = cptr - nfull * _CHUNK
        rb = nfull * _CHUNK
        for kk in range(_CHUNK // 16):
          dst_sl = pl.ds(kk * 16, 16)
          src_sl = pl.ds(rb + kk * 16, 16)
          csrc[dst_sl] = csrc[src_sl]
          cdst[dst_sl] = cdst[src_sl]
        return rem

      cptr = lax.fori_loop(0, _NBLK, block_body, 0)
      neg1 = jnp.full((16,), -1, jnp.int32)
      for kk in range(_CHUNK // 16):
        csrc[pl.ds(cptr + kk * 16, 16)] = neg1
        cdst[pl.ds(cptr + kk * 16, 16)] = neg1
      drain((cptr + _CHUNK - 1) // _CHUNK)

      plsc.subcore_barrier()

      @pl.when(s < _NS - 1)
      def _():
        pltpu.sync_copy(accs.at[pl.ds(r0, _STRIPE)],
                        agg_hbm.at[pl.ds(lo + r0, _STRIPE)])

      if p < _NP - 1:
        @pl.when(s == _NS - 1)
        def _():
          pltpu.sync_copy(accs.at[pl.ds(r0, _LAST)],
                          agg_hbm.at[pl.ds(lo + r0, _LAST)])
      else:
        @pl.when(jnp.logical_and(s == _NS - 1, c == 0))
        def _():
          pltpu.sync_copy(accs.at[pl.ds(r0, _LAST)],
                          agg_hbm.at[pl.ds(lo + r0, _LAST)])

        @pl.when(jnp.logical_and(s == _NS - 1, c == 1))
        def _():
          pltpu.sync_copy(accs.at[pl.ds(r0, _LAST5)],
                          agg_hbm.at[pl.ds(lo + r0, _LAST5)])

      if with_deg:
        def _deg_out(nrows):
          for kz in range(nrows // _CHUNK):
            off = r0 + kz * _CHUNK
            pltpu.sync_copy(degs.at[pl.ds(off, _CHUNK)], z1buf)
            pltpu.sync_copy(z1buf, deg_hbm.at[pl.ds(lo + off, _CHUNK)])
          tail = nrows % _CHUNK
          if tail:
            off = r0 + (nrows // _CHUNK) * _CHUNK
            pltpu.sync_copy(degs.at[pl.ds(off, tail)],
                            z1buf.at[pl.ds(0, tail)])
            pltpu.sync_copy(z1buf.at[pl.ds(0, tail)],
                            deg_hbm.at[pl.ds(lo + off, tail)])

        @pl.when(s < _NS - 1)
        def _():
          _deg_out(_STRIPE)

        if p < _NP - 1:
          @pl.when(s == _NS - 1)
          def _():
            _deg_out(_LAST)
        else:
          @pl.when(jnp.logical_and(s == _NS - 1, c == 0))
          def _():
            _deg_out(_LAST)

          @pl.when(jnp.logical_and(s == _NS - 1, c == 1))
          def _():
            _deg_out(_LAST5)

  return pl.kernel(
      body, out_type=out_type, mesh=mesh, scratch_types=scratch,
      compiler_params=pltpu.CompilerParams(needs_layout_passes=False))


_B = 2000


def _emb_body(x_ref, wemb_ref, bemb_ref, wl0_ref, h_ref, y_ref):
  h = jnp.maximum(
      jnp.dot(x_ref[...], wemb_ref[...], preferred_element_type=jnp.float32)
      + bemb_ref[...], 0.0)
  h_ref[...] = h
  y_ref[...] = jnp.dot(h, wl0_ref[...], preferred_element_type=jnp.float32)


def _emb_call(x, wemb, bemb, wl0p):
  return pl.pallas_call(
      _emb_body,
      grid=(_N // _B,),
      in_specs=[
          pl.BlockSpec((_B, 100), lambda i: (i, 0)),
          pl.BlockSpec((100, 128), lambda i: (0, 0)),
          pl.BlockSpec((1, 128), lambda i: (0, 0)),
          pl.BlockSpec((128, 128), lambda i: (0, 0)),
      ],
      out_specs=[
          pl.BlockSpec((_B, 128), lambda i: (i, 0)),
          pl.BlockSpec((_B, 128), lambda i: (i, 0)),
      ],
      out_shape=[
          jax.ShapeDtypeStruct((_N, 128), jnp.float32),
          jax.ShapeDtypeStruct((_N, 128), jnp.float32),
      ],
  )(x, wemb, bemb, wl0p)


def _mid_body(agg_ref, deg_ref, h_ref, wr_ref, bl_ref, wln_ref,
              h_out_ref, y_ref):
  deg = jnp.maximum(deg_ref[...], 1.0)
  mean = agg_ref[...][:, :80] / deg
  t = (mean + bl_ref[...]
       + jnp.dot(h_ref[...], wr_ref[...], preferred_element_type=jnp.float32))
  nrm = jnp.sqrt(jnp.sum(t * t, axis=1, keepdims=True))
  hn = jnp.maximum(t / jnp.maximum(nrm, 1e-12), 0.0)
  h_out_ref[...] = hn
  y_ref[...] = jnp.dot(hn, wln_ref[...], preferred_element_type=jnp.float32)


def _mid_call(agg, deg2d, h, wr, bl, wlnp, din):
  return pl.pallas_call(
      _mid_body,
      grid=(_N // _B,),
      in_specs=[
          pl.BlockSpec((_B, 128), lambda i: (i, 0)),
          pl.BlockSpec((_B, 1), lambda i: (i, 0)),
          pl.BlockSpec((_B, din), lambda i: (i, 0)),
          pl.BlockSpec((din, 80), lambda i: (0, 0)),
          pl.BlockSpec((1, 80), lambda i: (0, 0)),
          pl.BlockSpec((80, 128), lambda i: (0, 0)),
      ],
      out_specs=[
          pl.BlockSpec((_B, 80), lambda i: (i, 0)),
          pl.BlockSpec((_B, 128), lambda i: (i, 0)),
      ],
      out_shape=[
          jax.ShapeDtypeStruct((_N, 80), jnp.float32),
          jax.ShapeDtypeStruct((_N, 128), jnp.float32),
      ],
  )(agg, deg2d, h, wr, bl, wlnp)


def _final_body(agg_ref, deg_ref, h_ref, wr_ref, bl_ref, out_ref):
  deg = jnp.maximum(deg_ref[...], 1.0)
  mean = agg_ref[...][:, :18] / deg
  t = (mean + bl_ref[...]
       + jnp.dot(h_ref[...], wr_ref[...], preferred_element_type=jnp.float32))
  nrm = jnp.sqrt(jnp.sum(t * t, axis=1, keepdims=True))
  out_ref[...] = t / jnp.maximum(nrm, 1e-12)


def _final_call(agg, deg2d, h, wr, bl):
  return pl.pallas_call(
      _final_body,
      grid=(_N // _B,),
      in_specs=[
          pl.BlockSpec((_B, 128), lambda i: (i, 0)),
          pl.BlockSpec((_B, 1), lambda i: (i, 0)),
          pl.BlockSpec((_B, 80), lambda i: (i, 0)),
          pl.BlockSpec((80, 18), lambda i: (0, 0)),
          pl.BlockSpec((1, 18), lambda i: (0, 0)),
      ],
      out_specs=pl.BlockSpec((_B, 18), lambda i: (i, 0)),
      out_shape=jax.ShapeDtypeStruct((_N, 18), jnp.float32),
  )(agg, deg2d, h, wr, bl)


def _pad_cols(w, cols):
  return jnp.pad(w, ((0, 0), (0, cols - w.shape[1])))


def kernel(x, edge_index, W_emb, b_emb,
           W_l0, b_l0, W_r0,
           W_l1, b_l1, W_r1,
           W_l2, b_l2, W_r2,
           W_l3, b_l3, W_r3,
           W_l4, b_l4, W_r4):
  pad = _EPAD - _E
  srcp = jnp.concatenate(
      [edge_index[0], jnp.zeros((pad,), jnp.int32)]).reshape(_ROWS, _CHUNK)
  dstp = jnp.concatenate(
      [edge_index[1], jnp.full((pad,), _N, jnp.int32)]).reshape(_ROWS, _CHUNK)

  zacc = jnp.zeros((_CHUNK, 128), jnp.float32)
  zdeg = jnp.zeros((_CHUNK,), jnp.float32)
  ones = jnp.ones((_CHUNK,), jnp.float32)

  wl0p = _pad_cols(W_l0, 128)
  wl1p = _pad_cols(W_l1, 128)
  wl2p = _pad_cols(W_l2, 128)
  wl3p = _pad_cols(W_l3, 128)
  wl4p = _pad_cols(W_l4, 128)

  sc_deg = _make_sc_agg(True)
  sc = _make_sc_agg(False)

  h0, y0 = _emb_call(x, W_emb, b_emb.reshape(1, 128), wl0p)
  agg0, deg = sc_deg(y0, srcp, dstp, zacc, ones, zdeg)
  deg2d = deg.reshape(_N, 1)

  h1, y1 = _mid_call(agg0, deg2d, h0, W_r0, b_l0.reshape(1, 80), wl1p, 128)
  agg1 = sc(y1, srcp, dstp, zacc)
  h2, y2 = _mid_call(agg1, deg2d, h1, W_r1, b_l1.reshape(1, 80), wl2p, 80)
  agg2 = sc(y2, srcp, dstp, zacc)
  h3, y3 = _mid_call(agg2, deg2d, h2, W_r2, b_l2.reshape(1, 80), wl3p, 80)
  agg3 = sc(y3, srcp, dstp, zacc)
  h4, y4 = _mid_call(agg3, deg2d, h3, W_r3, b_l3.reshape(1, 80), wl4p, 80)
  agg4 = sc(y4, srcp, dstp, zacc)

  return _final_call(agg4, deg2d, h4, W_r4, b_l4.reshape(1, 18))


# speedup vs baseline: 9.8308x; 1.6460x over previous
import functools

import jax
import jax.numpy as jnp
from jax import lax
from jax.experimental import pallas as pl
from jax.experimental.pallas import tpu as pltpu
from jax.experimental.pallas import tpu_sc as plsc

_N = 50000
_E = 800000
_NS = 16
_CHUNK = 128
_EPAD = 835584
_ROWS = _EPAD // _CHUNK
_RPT = _ROWS // _NS
_BLK = 24
_NBLK = _RPT // _BLK
_NP = 3
_Q = 8336
_ACC_ROWS = 8448
_STRIPE = _ACC_ROWS // _NS
_LAST = _Q - 15 * _STRIPE
_LAST5 = (_N - 5 * _Q) - 15 * _STRIPE
_BCAP = 432
_RTROWS = 6 * _NS * _BCAP
_FCAP = 1280
_FLUSH = 1024


def _make_router():
  mesh = plsc.VectorSubcoreMesh(core_axis_name="c", subcore_axis_name="s")
  out_type = [
      jax.ShapeDtypeStruct((_RTROWS, _CHUNK), jnp.int32),
      jax.ShapeDtypeStruct((_RTROWS, _CHUNK), jnp.int32),
      jax.ShapeDtypeStruct((2 * _NS * 16,), jnp.int32),
  ]
  scratch = [
      pltpu.VMEM((_BLK, _CHUNK), jnp.int32),
      pltpu.VMEM((_BLK, _CHUNK), jnp.int32),
      pltpu.VMEM((_FCAP,), jnp.int32),
      pltpu.VMEM((_FCAP,), jnp.int32),
      pltpu.VMEM((_FCAP,), jnp.int32),
      pltpu.VMEM((_FCAP,), jnp.int32),
      pltpu.VMEM((_FCAP,), jnp.int32),
      pltpu.VMEM((_FCAP,), jnp.int32),
      pltpu.VMEM((8, _CHUNK), jnp.int32),
      pltpu.VMEM((16,), jnp.int32),
  ]

  def body(src_hbm, dst_hbm, rsrc_hbm, rdst_hbm, cnt_hbm,
           srcv, dstv, cs0, cs1, cs2, cd0, cd1, cd2, bounce, cntv):
    c = lax.axis_index("c")
    s = lax.axis_index("s")
    w = c * _NS + s
    csl = (cs0, cs1, cs2)
    cdl = (cd0, cd1, cd2)
    neg1 = jnp.full((16,), -1, jnp.int32)

    def bucket_row(p):
      return ((c * 3 + p) * _NS + s) * _BCAP

    def block_body(blk, carry):
      cp = list(carry[0:3])
      rw = list(carry[3:6])
      base = s * _RPT + blk * _BLK
      pltpu.sync_copy(src_hbm.at[pl.ds(base, _BLK)], srcv)
      pltpu.sync_copy(dst_hbm.at[pl.ds(base, _BLK)], dstv)

      def row(j, rcarry):
        cp = list(rcarry[0:3])
        rw = list(rcarry[3:6])
        for kk in range(_CHUNK // 16):
          sl = pl.ds(kk * 16, 16)
          sv = srcv[j, sl]
          dv = dstv[j, sl]
          for p in range(3):
            lo = (c * 3 + p) * _Q
            hi = jnp.minimum(lo + _Q, _N)
            mine = jnp.logical_and(dv >= lo, dv < hi)
            mi = jnp.where(mine, 1, 0)
            _, ssrc = plsc.sort_key_val(mi, jnp.where(mine, sv, -1),
                                        descending=True)
            _, sdst = plsc.sort_key_val(mi, jnp.where(mine, dv - lo, -1),
                                        descending=True)
            csl[p][pl.ds(cp[p], 16)] = ssrc
            cdl[p][pl.ds(cp[p], 16)] = sdst
            cp[p] = cp[p] + plsc.all_reduce_population_count(mine)[0]
        for p in range(3):
          fl = cp[p] >= _FLUSH

          @pl.when(fl)
          def _():
            for r in range(8):
              for g in range(8):
                gsl = pl.ds(g * 16, 16)
                bounce[r, gsl] = csl[p][pl.ds(r * _CHUNK + g * 16, 16)]
            pltpu.sync_copy(bounce, rsrc_hbm.at[
                pl.ds(pl.multiple_of(bucket_row(p) + rw[p], 8), 8)])
            for r in range(8):
              for g in range(8):
                gsl = pl.ds(g * 16, 16)
                bounce[r, gsl] = cdl[p][pl.ds(r * _CHUNK + g * 16, 16)]
            pltpu.sync_copy(bounce, rdst_hbm.at[
                pl.ds(pl.multiple_of(bucket_row(p) + rw[p], 8), 8)])
            for g in range(8):
              gsl = pl.ds(g * 16, 16)
              csl[p][gsl] = csl[p][pl.ds(_FLUSH + g * 16, 16)]
              cdl[p][gsl] = cdl[p][pl.ds(_FLUSH + g * 16, 16)]

          cp[p] = jnp.where(fl, cp[p] - _FLUSH, cp[p])
          rw[p] = jnp.where(fl, rw[p] + 8, rw[p])
        return (*cp, *rw)

      return lax.fori_loop(0, _BLK, row, (*cp, *rw))

    zero = jnp.zeros((), jnp.int32)
    carry = lax.fori_loop(0, _NBLK, block_body,
                          (zero, zero, zero, zero, zero, zero))
    cp = list(carry[0:3])
    rw = list(carry[3:6])

    lane = lax.iota(jnp.int32, 16)
    cvec = jnp.zeros((16,), jnp.int32)
    for p in range(3):
      for kk in range(_CHUNK // 16):
        csl[p][pl.ds(cp[p] + kk * 16, 16)] = neg1
        cdl[p][pl.ds(cp[p] + kk * 16, 16)] = neg1
      for r in range(8):
        ok = r * _CHUNK < cp[p] + _CHUNK
        for g in range(8):
          gsl = pl.ds(g * 16, 16)
          v = csl[p][pl.ds(r * _CHUNK + g * 16, 16)]
          bounce[r, gsl] = jnp.where(ok, v, neg1)
      pltpu.sync_copy(bounce, rsrc_hbm.at[
          pl.ds(pl.multiple_of(bucket_row(p) + rw[p], 8), 8)])
      for r in range(8):
        ok = r * _CHUNK < cp[p] + _CHUNK
        for g in range(8):
          gsl = pl.ds(g * 16, 16)
          v = cdl[p][pl.ds(r * _CHUNK + g * 16, 16)]
          bounce[r, gsl] = jnp.where(ok, v, neg1)
      pltpu.sync_copy(bounce, rdst_hbm.at[
          pl.ds(pl.multiple_of(bucket_row(p) + rw[p], 8), 8)])
      nch = rw[p] + (cp[p] + _CHUNK - 1) // _CHUNK
      cvec = jnp.where(lane == p, nch, cvec)
    cntv[...] = cvec
    pltpu.sync_copy(cntv, cnt_hbm.at[pl.ds(w * 16, 16)])

  return pl.kernel(
      body, out_type=out_type, mesh=mesh, scratch_types=scratch,
      compiler_params=pltpu.CompilerParams(needs_layout_passes=False))


def _make_sc_agg(with_deg):
  mesh = plsc.VectorSubcoreMesh(core_axis_name="c", subcore_axis_name="s")

  if with_deg:
    out_type = [jax.ShapeDtypeStruct((_N, 128), jnp.float32),
                jax.ShapeDtypeStruct((_N,), jnp.float32)]
  else:
    out_type = jax.ShapeDtypeStruct((_N, 128), jnp.float32)

  scratch = [
      pltpu.VMEM((_BLK, _CHUNK), jnp.int32),
      pltpu.VMEM((_BLK, _CHUNK), jnp.int32),
      pltpu.VMEM((16,), jnp.int32),
      pltpu.VMEM((_CHUNK, 128), jnp.float32),
      pltpu.VMEM((_CHUNK, 128), jnp.float32),
      pltpu.VMEM_SHARED((_ACC_ROWS, 128), jnp.float32),
      pltpu.SemaphoreType.DMA,
      pltpu.SemaphoreType.DMA,
  ]
  if with_deg:
    scratch += [
        pltpu.VMEM((_CHUNK,), jnp.float32),
        pltpu.VMEM((_CHUNK,), jnp.float32),
        pltpu.VMEM_SHARED((_ACC_ROWS,), jnp.float32),
    ]

  def body(y_hbm, rsrc_hbm, rdst_hbm, cnt_hbm, zacc_hbm, *rest):
    if with_deg:
      (ones_hbm, zdeg_hbm, agg_hbm, deg_hbm,
       srcv, dstv, cntv, g0, g1, accs, sem0, sem1,
       onesv, z1buf, degs) = rest
    else:
      agg_hbm, srcv, dstv, cntv, g0, g1, accs, sem0, sem1 = rest

    c = lax.axis_index("c")
    s = lax.axis_index("s")
    w = c * _NS + s

    pltpu.sync_copy(cnt_hbm.at[pl.ds(w * 16, 16)], cntv)
    if with_deg:
      pltpu.sync_copy(ones_hbm, onesv)
    counts = cntv[...]

    r0 = s * _STRIPE

    for p in range(_NP):
      q = c * _NP + p
      lo = q * _Q
      hi = jnp.minimum(lo + _Q, _N)

      pltpu.sync_copy(zacc_hbm, g0)
      for kz in range(_STRIPE // _CHUNK):
        pltpu.sync_copy(g0, accs.at[pl.ds(r0 + kz * _CHUNK, _CHUNK)])
      ztail = _STRIPE % _CHUNK
      if ztail:
        pltpu.sync_copy(g0.at[pl.ds(0, ztail)],
                        accs.at[pl.ds(r0 + (_STRIPE // _CHUNK) * _CHUNK,
                                      ztail)])
      if with_deg:
        pltpu.sync_copy(zdeg_hbm, z1buf)
        for kz in range(_STRIPE // _CHUNK):
          pltpu.sync_copy(z1buf, degs.at[pl.ds(r0 + kz * _CHUNK, _CHUNK)])
        if ztail:
          pltpu.sync_copy(z1buf.at[pl.ds(0, ztail)],
                          degs.at[pl.ds(r0 + (_STRIPE // _CHUNK) * _CHUNK,
                                        ztail)])

      plsc.subcore_barrier()

      base_row = ((c * 3 + p) * _NS + s) * _BCAP
      nch = counts[p]

      def _gi(j):
        return plsc.Indices(srcv.at[j], ignored_value=-1)

      def _di(j):
        return plsc.Indices(dstv.at[j], ignored_value=-1)

      def drain(n):
        @pl.when(n > 0)
        def _():
          pltpu.async_copy(y_hbm.at[_gi(0)], g0, sem0)

        @pl.when(n > 1)
        def _():
          pltpu.async_copy(y_hbm.at[_gi(1)], g1, sem1)

        def pair(t, carry):
          j0 = t * 2
          pltpu.make_async_copy(y_hbm.at[_gi(j0)], g0, sem0).wait()
          pltpu.sync_copy(g0, accs.at[_di(j0)], add=True)
          if with_deg:
            pltpu.sync_copy(onesv, degs.at[_di(j0)], add=True)

          @pl.when(j0 + 2 < n)
          def _():
            pltpu.async_copy(y_hbm.at[_gi(j0 + 2)], g0, sem0)

          j1 = j0 + 1

          @pl.when(j1 < n)
          def _():
            pltpu.make_async_copy(y_hbm.at[_gi(j1)], g1, sem1).wait()
            pltpu.sync_copy(g1, accs.at[_di(j1)], add=True)
            if with_deg:
              pltpu.sync_copy(onesv, degs.at[_di(j1)], add=True)

            @pl.when(j1 + 2 < n)
            def _():
              pltpu.async_copy(y_hbm.at[_gi(j1 + 2)], g1, sem1)

          return carry

        lax.fori_loop(0, (n + 1) // 2, pair, 0)

      def block_body(blk, carry):
        pltpu.sync_copy(rsrc_hbm.at[
            pl.ds(pl.multiple_of(base_row + blk * _BLK, 8), _BLK)], srcv)
        pltpu.sync_copy(rdst_hbm.at[
            pl.ds(pl.multiple_of(base_row + blk * _BLK, 8), _BLK)], dstv)
        drain(jnp.minimum(_BLK, nch - blk * _BLK))
        return carry

      lax.fori_loop(0, (nch + _BLK - 1) // _BLK, block_body, 0)

      plsc.subcore_barrier()

      @pl.when(s < _NS - 1)
      def _():
        pltpu.sync_copy(accs.at[pl.ds(r0, _STRIPE)],
                        agg_hbm.at[pl.ds(lo + r0, _STRIPE)])

      if p < _NP - 1:
        @pl.when(s == _NS - 1)
        def _():
          pltpu.sync_copy(accs.at[pl.ds(r0, _LAST)],
                          agg_hbm.at[pl.ds(lo + r0, _LAST)])
      else:
        @pl.when(jnp.logical_and(s == _NS - 1, c == 0))
        def _():
          pltpu.sync_copy(accs.at[pl.ds(r0, _LAST)],
                          agg_hbm.at[pl.ds(lo + r0, _LAST)])

        @pl.when(jnp.logical_and(s == _NS - 1, c == 1))
        def _():
          pltpu.sync_copy(accs.at[pl.ds(r0, _LAST5)],
                          agg_hbm.at[pl.ds(lo + r0, _LAST5)])

      if with_deg:
        def _deg_out(nrows):
          for kz in range(nrows // _CHUNK):
            off = r0 + kz * _CHUNK
            pltpu.sync_copy(degs.at[pl.ds(off, _CHUNK)], z1buf)
            pltpu.sync_copy(z1buf, deg_hbm.at[pl.ds(lo + off, _CHUNK)])
          tail = nrows % _CHUNK
          if tail:
            off = r0 + (nrows // _CHUNK) * _CHUNK
            pltpu.sync_copy(degs.at[pl.ds(off, tail)],
                            z1buf.at[pl.ds(0, tail)])
            pltpu.sync_copy(z1buf.at[pl.ds(0, tail)],
                            deg_hbm.at[pl.ds(lo + off, tail)])

        @pl.when(s < _NS - 1)
        def _():
          _deg_out(_STRIPE)

        if p < _NP - 1:
          @pl.when(s == _NS - 1)
          def _():
            _deg_out(_LAST)
        else:
          @pl.when(jnp.logical_and(s == _NS - 1, c == 0))
          def _():
            _deg_out(_LAST)

          @pl.when(jnp.logical_and(s == _NS - 1, c == 1))
          def _():
            _deg_out(_LAST5)

  return pl.kernel(
      body, out_type=out_type, mesh=mesh, scratch_types=scratch,
      compiler_params=pltpu.CompilerParams(needs_layout_passes=False))


_B = 2000


def _emb_body(x_ref, wemb_ref, bemb_ref, wl0_ref, h_ref, y_ref):
  h = jnp.maximum(
      jnp.dot(x_ref[...], wemb_ref[...], preferred_element_type=jnp.float32)
      + bemb_ref[...], 0.0)
  h_ref[...] = h
  y_ref[...] = jnp.dot(h, wl0_ref[...], preferred_element_type=jnp.float32)


def _emb_call(x, wemb, bemb, wl0p):
  return pl.pallas_call(
      _emb_body,
      grid=(_N // _B,),
      in_specs=[
          pl.BlockSpec((_B, 100), lambda i: (i, 0)),
          pl.BlockSpec((100, 128), lambda i: (0, 0)),
          pl.BlockSpec((1, 128), lambda i: (0, 0)),
          pl.BlockSpec((128, 128), lambda i: (0, 0)),
      ],
      out_specs=[
          pl.BlockSpec((_B, 128), lambda i: (i, 0)),
          pl.BlockSpec((_B, 128), lambda i: (i, 0)),
      ],
      out_shape=[
          jax.ShapeDtypeStruct((_N, 128), jnp.float32),
          jax.ShapeDtypeStruct((_N, 128), jnp.float32),
      ],
  )(x, wemb, bemb, wl0p)


def _mid_body(agg_ref, deg_ref, h_ref, wr_ref, bl_ref, wln_ref,
              h_out_ref, y_ref):
  deg = jnp.maximum(deg_ref[...], 1.0)
  mean = agg_ref[...][:, :80] / deg
  t = (mean + bl_ref[...]
       + jnp.dot(h_ref[...], wr_ref[...], preferred_element_type=jnp.float32))
  nrm = jnp.sqrt(jnp.sum(t * t, axis=1, keepdims=True))
  hn = jnp.maximum(t / jnp.maximum(nrm, 1e-12), 0.0)
  h_out_ref[...] = hn
  y_ref[...] = jnp.dot(hn, wln_ref[...], preferred_element_type=jnp.float32)


def _mid_call(agg, deg2d, h, wr, bl, wlnp, din):
  return pl.pallas_call(
      _mid_body,
      grid=(_N // _B,),
      in_specs=[
          pl.BlockSpec((_B, 128), lambda i: (i, 0)),
          pl.BlockSpec((_B, 1), lambda i: (i, 0)),
          pl.BlockSpec((_B, din), lambda i: (i, 0)),
          pl.BlockSpec((din, 80), lambda i: (0, 0)),
          pl.BlockSpec((1, 80), lambda i: (0, 0)),
          pl.BlockSpec((80, 128), lambda i: (0, 0)),
      ],
      out_specs=[
          pl.BlockSpec((_B, 80), lambda i: (i, 0)),
          pl.BlockSpec((_B, 128), lambda i: (i, 0)),
      ],
      out_shape=[
          jax.ShapeDtypeStruct((_N, 80), jnp.float32),
          jax.ShapeDtypeStruct((_N, 128), jnp.float32),
      ],
  )(agg, deg2d, h, wr, bl, wlnp)


def _final_body(agg_ref, deg_ref, h_ref, wr_ref, bl_ref, out_ref):
  deg = jnp.maximum(deg_ref[...], 1.0)
  mean = agg_ref[...][:, :18] / deg
  t = (mean + bl_ref[...]
       + jnp.dot(h_ref[...], wr_ref[...], preferred_element_type=jnp.float32))
  nrm = jnp.sqrt(jnp.sum(t * t, axis=1, keepdims=True))
  out_ref[...] = t / jnp.maximum(nrm, 1e-12)


def _final_call(agg, deg2d, h, wr, bl):
  return pl.pallas_call(
      _final_body,
      grid=(_N // _B,),
      in_specs=[
          pl.BlockSpec((_B, 128), lambda i: (i, 0)),
          pl.BlockSpec((_B, 1), lambda i: (i, 0)),
          pl.BlockSpec((_B, 80), lambda i: (i, 0)),
          pl.BlockSpec((80, 18), lambda i: (0, 0)),
          pl.BlockSpec((1, 18), lambda i: (0, 0)),
      ],
      out_specs=pl.BlockSpec((_B, 18), lambda i: (i, 0)),
      out_shape=jax.ShapeDtypeStruct((_N, 18), jnp.float32),
  )(agg, deg2d, h, wr, bl)


def _pad_cols(w, cols):
  return jnp.pad(w, ((0, 0), (0, cols - w.shape[1])))


def kernel(x, edge_index, W_emb, b_emb,
           W_l0, b_l0, W_r0,
           W_l1, b_l1, W_r1,
           W_l2, b_l2, W_r2,
           W_l3, b_l3, W_r3,
           W_l4, b_l4, W_r4):
  pad = _EPAD - _E
  srcp = jnp.concatenate(
      [edge_index[0], jnp.zeros((pad,), jnp.int32)]).reshape(_ROWS, _CHUNK)
  dstp = jnp.concatenate(
      [edge_index[1], jnp.full((pad,), _N, jnp.int32)]).reshape(_ROWS, _CHUNK)

  zacc = jnp.zeros((_CHUNK, 128), jnp.float32)
  zdeg = jnp.zeros((_CHUNK,), jnp.float32)
  ones = jnp.ones((_CHUNK,), jnp.float32)

  wl0p = _pad_cols(W_l0, 128)
  wl1p = _pad_cols(W_l1, 128)
  wl2p = _pad_cols(W_l2, 128)
  wl3p = _pad_cols(W_l3, 128)
  wl4p = _pad_cols(W_l4, 128)

  router = _make_router()
  sc_deg = _make_sc_agg(True)
  sc = _make_sc_agg(False)

  rsrc, rdst, cnts = router(srcp, dstp)

  h0, y0 = _emb_call(x, W_emb, b_emb.reshape(1, 128), wl0p)
  agg0, deg = sc_deg(y0, rsrc, rdst, cnts, zacc, ones, zdeg)
  deg2d = deg.reshape(_N, 1)

  h1, y1 = _mid_call(agg0, deg2d, h0, W_r0, b_l0.reshape(1, 80), wl1p, 128)
  agg1 = sc(y1, rsrc, rdst, cnts, zacc)
  h2, y2 = _mid_call(agg1, deg2d, h1, W_r1, b_l1.reshape(1, 80), wl2p, 80)
  agg2 = sc(y2, rsrc, rdst, cnts, zacc)
  h3, y3 = _mid_call(agg2, deg2d, h2, W_r2, b_l2.reshape(1, 80), wl3p, 80)
  agg3 = sc(y3, rsrc, rdst, cnts, zacc)
  h4, y4 = _mid_call(agg3, deg2d, h3, W_r3, b_l3.reshape(1, 80), wl4p, 80)
  agg4 = sc(y4, rsrc, rdst, cnts, zacc)

  return _final_call(agg4, deg2d, h4, W_r4, b_l4.reshape(1, 18))
